# baseline (speedup 1.0000x reference)
import functools

import jax
import jax.numpy as jnp
from jax import lax
from jax.experimental import pallas as pl
from jax.experimental.pallas import tpu as pltpu
from jax.experimental.pallas import tpu_sc as plsc

VOCAB = 100000
VOCAB_PAD = 102400
EMBED_DIM = 64
NUM_CLASS = 2
BATCH = 4096
SEQ = 200

NUM_CORES = 2
NUM_SUBCORES = 16
NUM_WORKERS = NUM_CORES * NUM_SUBCORES
B_PER_W = BATCH // NUM_WORKERS
R_CHUNKS = B_PER_W // 16
TOK_PER_W = B_PER_W * SEQ


def _tc_project_body(w_ref, b_ref, emb_ref, out_ref):
    d = (
        lax.dot_general(
            w_ref[...],
            emb_ref[...],
            dimension_numbers=(((1,), (1,)), ((), ())),
            preferred_element_type=jnp.float32,
        )
        + b_ref[...]
    )
    u0 = lax.bitcast_convert_type(d[0:1, :].astype(jnp.bfloat16), jnp.uint16)
    u1 = lax.bitcast_convert_type(d[1:2, :].astype(jnp.bfloat16), jnp.uint16)
    packed = u0.astype(jnp.uint32) | (u1.astype(jnp.uint32) << 16)
    out_ref[...] = lax.bitcast_convert_type(packed, jnp.int32).reshape(-1)


_TC_GRID = 25

_tc_project = pl.pallas_call(
    _tc_project_body,
    grid=(_TC_GRID,),
    in_specs=[
        pl.BlockSpec((NUM_CLASS, EMBED_DIM), lambda i: (0, 0)),
        pl.BlockSpec((NUM_CLASS, 1), lambda i: (0, 0)),
        pl.BlockSpec((VOCAB_PAD // _TC_GRID, EMBED_DIM), lambda i: (i, 0)),
    ],
    out_specs=pl.BlockSpec((VOCAB_PAD // _TC_GRID,), lambda i: (i,)),
    out_shape=jax.ShapeDtypeStruct((VOCAB_PAD,), jnp.int32),
)


@functools.lru_cache(maxsize=1)
def _make_sc_pool():
    mesh = plsc.VectorSubcoreMesh(
        core_axis_name="c",
        subcore_axis_name="s",
        num_cores=NUM_CORES,
        num_subcores=NUM_SUBCORES,
    )

    @functools.partial(
        pl.kernel,
        mesh=mesh,
        out_type=jax.ShapeDtypeStruct((NUM_CLASS * BATCH,), jnp.float32),
        scratch_types=[
            pltpu.VMEM((TOK_PER_W,), jnp.int32),
            pltpu.VMEM((VOCAB,), jnp.int32),
            pltpu.VMEM((NUM_CLASS * B_PER_W,), jnp.float32),
            pltpu.SemaphoreType.DMA,
            pltpu.SemaphoreType.DMA,
        ],
        compiler_params=pltpu.CompilerParams(needs_layout_passes=False),
    )
    def _sc_pool(ptab_hbm, text_hbm, out_hbm, idx_v, tab_v, out_v, sem1, sem2):
        wid = lax.axis_index("s") * NUM_CORES + lax.axis_index("c")
        base = wid * B_PER_W
        c_tab = pltpu.async_copy(ptab_hbm.at[pl.ds(0, VOCAB)], tab_v, sem1)
        c_idx = pltpu.async_copy(
            text_hbm.at[pl.ds(base * SEQ, TOK_PER_W)], idx_v, sem2
        )
        c_idx.wait()
        c_tab.wait()

        lanevec = lax.iota(jnp.int32, 16) * SEQ
        himask = jnp.full((16,), -65536, jnp.int32)

        def body(p, accs):
            out = []
            for r in range(R_CHUNKS):
                addr = lanevec + (r * 16 * SEQ + p)
                iv = plsc.load_gather(idx_v, [addr])
                g = plsc.load_gather(tab_v, [iv])
                f0 = plsc.bitcast(g << 16, jnp.float32)
                f1 = plsc.bitcast(g & himask, jnp.float32)
                out.append(accs[2 * r] + f0)
                out.append(accs[2 * r + 1] + f1)
            return tuple(out)

        zero = jnp.zeros((16,), jnp.float32)
        accs = lax.fori_loop(
            0, SEQ, body, (zero,) * (2 * R_CHUNKS), unroll=2
        )
        for r in range(R_CHUNKS):
            out_v[pl.ds(r * 16, 16)] = accs[2 * r] * (1.0 / SEQ)
            out_v[pl.ds(B_PER_W + r * 16, 16)] = accs[2 * r + 1] * (1.0 / SEQ)
        c_o0 = pltpu.async_copy(
            out_v.at[pl.ds(0, B_PER_W)], out_hbm.at[pl.ds(base, B_PER_W)], sem1
        )
        c_o1 = pltpu.async_copy(
            out_v.at[pl.ds(B_PER_W, B_PER_W)],
            out_hbm.at[pl.ds(BATCH + base, B_PER_W)],
            sem2,
        )
        c_o0.wait()
        c_o1.wait()

    return _sc_pool


def kernel(text, emb_table, fc_w, fc_b):
    text = text.astype(jnp.int32).reshape(-1)
    ptab = _tc_project(fc_w, fc_b.reshape(NUM_CLASS, 1), emb_table)
    out = _make_sc_pool()(ptab, text)
    return out.reshape(NUM_CLASS, BATCH).T


# speedup vs baseline: 1.0864x; 1.0864x over previous
import functools

import jax
import jax.numpy as jnp
from jax import lax
from jax.experimental import pallas as pl
from jax.experimental.pallas import tpu as pltpu
from jax.experimental.pallas import tpu_sc as plsc

VOCAB = 100000
VOCAB_PAD = 102400
EMBED_DIM = 64
NUM_CLASS = 2
BATCH = 4096
SEQ = 200

NUM_CORES = 2
NUM_SUBCORES = 16
NUM_WORKERS = NUM_CORES * NUM_SUBCORES
B_PER_W = BATCH // NUM_WORKERS
R_CHUNKS = B_PER_W // 16
TOK_PER_W = B_PER_W * SEQ


def _tc_project_body(w_ref, b_ref, emb_ref, out_ref):
    d = (
        lax.dot_general(
            w_ref[...],
            emb_ref[...],
            dimension_numbers=(((1,), (1,)), ((), ())),
            preferred_element_type=jnp.float32,
        )
        + b_ref[...]
    )
    u0 = lax.bitcast_convert_type(d[0:1, :].astype(jnp.bfloat16), jnp.uint16)
    u1 = lax.bitcast_convert_type(d[1:2, :].astype(jnp.bfloat16), jnp.uint16)
    packed = u0.astype(jnp.uint32) | (u1.astype(jnp.uint32) << 16)
    out_ref[...] = lax.bitcast_convert_type(packed, jnp.int32).reshape(-1)


_TC_GRID = 10

_tc_project = pl.pallas_call(
    _tc_project_body,
    grid=(_TC_GRID,),
    in_specs=[
        pl.BlockSpec((NUM_CLASS, EMBED_DIM), lambda i: (0, 0)),
        pl.BlockSpec((NUM_CLASS, 1), lambda i: (0, 0)),
        pl.BlockSpec((VOCAB_PAD // _TC_GRID, EMBED_DIM), lambda i: (i, 0)),
    ],
    out_specs=pl.BlockSpec((VOCAB_PAD // _TC_GRID,), lambda i: (i,)),
    out_shape=jax.ShapeDtypeStruct((VOCAB_PAD,), jnp.int32),
)


@functools.lru_cache(maxsize=1)
def _make_sc_pool():
    mesh = plsc.VectorSubcoreMesh(
        core_axis_name="c",
        subcore_axis_name="s",
        num_cores=NUM_CORES,
        num_subcores=NUM_SUBCORES,
    )

    @functools.partial(
        pl.kernel,
        mesh=mesh,
        out_type=jax.ShapeDtypeStruct((NUM_CLASS * BATCH,), jnp.float32),
        scratch_types=[
            pltpu.VMEM((TOK_PER_W,), jnp.int32),
            pltpu.VMEM((VOCAB,), jnp.int32),
            pltpu.VMEM((NUM_CLASS * B_PER_W,), jnp.float32),
            pltpu.SemaphoreType.DMA,
            pltpu.SemaphoreType.DMA,
        ],
        compiler_params=pltpu.CompilerParams(needs_layout_passes=False),
    )
    def _sc_pool(ptab_hbm, text_hbm, out_hbm, idx_v, tab_v, out_v, sem1, sem2):
        wid = lax.axis_index("s") * NUM_CORES + lax.axis_index("c")
        base = wid * B_PER_W
        c_tab = pltpu.async_copy(ptab_hbm.at[pl.ds(0, VOCAB)], tab_v, sem1)
        c_idx = pltpu.async_copy(
            text_hbm.at[pl.ds(base * SEQ, TOK_PER_W)], idx_v, sem2
        )
        c_idx.wait()
        c_tab.wait()

        lanevec = lax.iota(jnp.int32, 16) * SEQ
        himask = jnp.full((16,), -65536, jnp.int32)

        def body(p, accs):
            out = []
            for r in range(R_CHUNKS):
                addr = lanevec + (r * 16 * SEQ + p)
                iv = plsc.load_gather(idx_v, [addr])
                g = plsc.load_gather(tab_v, [iv])
                f0 = plsc.bitcast(g << 16, jnp.float32)
                f1 = plsc.bitcast(g & himask, jnp.float32)
                out.append(accs[2 * r] + f0)
                out.append(accs[2 * r + 1] + f1)
            return tuple(out)

        zero = jnp.zeros((16,), jnp.float32)
        accs = lax.fori_loop(
            0, SEQ, body, (zero,) * (2 * R_CHUNKS), unroll=2
        )
        for r in range(R_CHUNKS):
            out_v[pl.ds(r * 16, 16)] = accs[2 * r] * (1.0 / SEQ)
            out_v[pl.ds(B_PER_W + r * 16, 16)] = accs[2 * r + 1] * (1.0 / SEQ)
        c_o0 = pltpu.async_copy(
            out_v.at[pl.ds(0, B_PER_W)], out_hbm.at[pl.ds(base, B_PER_W)], sem1
        )
        c_o1 = pltpu.async_copy(
            out_v.at[pl.ds(B_PER_W, B_PER_W)],
            out_hbm.at[pl.ds(BATCH + base, B_PER_W)],
            sem2,
        )
        c_o0.wait()
        c_o1.wait()

    return _sc_pool


def kernel(text, emb_table, fc_w, fc_b):
    text = text.astype(jnp.int32).reshape(-1)
    ptab = _tc_project(fc_w, fc_b.reshape(NUM_CLASS, 1), emb_table)
    out = _make_sc_pool()(ptab, text)
    return out.reshape(NUM_CLASS, BATCH).T


# speedup vs baseline: 1.0930x; 1.0061x over previous
import functools

import jax
import jax.numpy as jnp
from jax import lax
from jax.experimental import pallas as pl
from jax.experimental.pallas import tpu as pltpu
from jax.experimental.pallas import tpu_sc as plsc

VOCAB = 100000
VOCAB_PAD = 102400
EMBED_DIM = 64
NUM_CLASS = 2
BATCH = 4096
SEQ = 200

NUM_CORES = 2
NUM_SUBCORES = 16
NUM_WORKERS = NUM_CORES * NUM_SUBCORES
B_PER_W = BATCH // NUM_WORKERS
R_CHUNKS = B_PER_W // 16
TOK_PER_W = B_PER_W * SEQ


def _tc_project_body(w_ref, b_ref, emb_ref, out_ref):
    d = (
        lax.dot_general(
            w_ref[...],
            emb_ref[...],
            dimension_numbers=(((1,), (1,)), ((), ())),
            preferred_element_type=jnp.float32,
        )
        + b_ref[...]
    )
    u0 = lax.bitcast_convert_type(d[0:1, :].astype(jnp.bfloat16), jnp.uint16)
    u1 = lax.bitcast_convert_type(d[1:2, :].astype(jnp.bfloat16), jnp.uint16)
    packed = u0.astype(jnp.uint32) | (u1.astype(jnp.uint32) << 16)
    out_ref[...] = lax.bitcast_convert_type(packed, jnp.int32)[None]


_tc_project = pl.pallas_call(
    _tc_project_body,
    grid=(8,),
    in_specs=[
        pl.BlockSpec((NUM_CLASS, EMBED_DIM), lambda i: (0, 0)),
        pl.BlockSpec((NUM_CLASS, 1), lambda i: (0, 0)),
        pl.BlockSpec((VOCAB_PAD // 8, EMBED_DIM), lambda i: (i, 0)),
    ],
    out_specs=pl.BlockSpec((1, 1, VOCAB_PAD // 8), lambda i: (i, 0, 0)),
    out_shape=jax.ShapeDtypeStruct((8, 1, VOCAB_PAD // 8), jnp.int32),
)


@functools.lru_cache(maxsize=1)
def _make_sc_pool():
    mesh = plsc.VectorSubcoreMesh(
        core_axis_name="c",
        subcore_axis_name="s",
        num_cores=NUM_CORES,
        num_subcores=NUM_SUBCORES,
    )

    @functools.partial(
        pl.kernel,
        mesh=mesh,
        out_type=jax.ShapeDtypeStruct((NUM_CLASS * BATCH,), jnp.float32),
        scratch_types=[
            pltpu.VMEM((TOK_PER_W,), jnp.int32),
            pltpu.VMEM((VOCAB,), jnp.int32),
            pltpu.VMEM((NUM_CLASS * B_PER_W,), jnp.float32),
            pltpu.SemaphoreType.DMA,
            pltpu.SemaphoreType.DMA,
        ],
        compiler_params=pltpu.CompilerParams(needs_layout_passes=False),
    )
    def _sc_pool(ptab_hbm, text_hbm, out_hbm, idx_v, tab_v, out_v, sem1, sem2):
        wid = lax.axis_index("s") * NUM_CORES + lax.axis_index("c")
        base = wid * B_PER_W
        c_tab = pltpu.async_copy(ptab_hbm.at[pl.ds(0, VOCAB)], tab_v, sem1)
        c_idx = pltpu.async_copy(
            text_hbm.at[pl.ds(base * SEQ, TOK_PER_W)], idx_v, sem2
        )
        c_idx.wait()
        c_tab.wait()

        lanevec = lax.iota(jnp.int32, 16) * SEQ
        himask = jnp.full((16,), -65536, jnp.int32)

        def body(p, accs):
            out = []
            for r in range(R_CHUNKS):
                addr = lanevec + (r * 16 * SEQ + p)
                iv = plsc.load_gather(idx_v, [addr])
                g = plsc.load_gather(tab_v, [iv])
                f0 = plsc.bitcast(g << 16, jnp.float32)
                f1 = plsc.bitcast(g & himask, jnp.float32)
                out.append(accs[2 * r] + f0)
                out.append(accs[2 * r + 1] + f1)
            return tuple(out)

        zero = jnp.zeros((16,), jnp.float32)
        accs = lax.fori_loop(
            0, SEQ, body, (zero,) * (2 * R_CHUNKS), unroll=2
        )
        for r in range(R_CHUNKS):
            out_v[pl.ds(r * 16, 16)] = accs[2 * r] * (1.0 / SEQ)
            out_v[pl.ds(B_PER_W + r * 16, 16)] = accs[2 * r + 1] * (1.0 / SEQ)
        c_o0 = pltpu.async_copy(
            out_v.at[pl.ds(0, B_PER_W)], out_hbm.at[pl.ds(base, B_PER_W)], sem1
        )
        c_o1 = pltpu.async_copy(
            out_v.at[pl.ds(B_PER_W, B_PER_W)],
            out_hbm.at[pl.ds(BATCH + base, B_PER_W)],
            sem2,
        )
        c_o0.wait()
        c_o1.wait()

    return _sc_pool


def kernel(text, emb_table, fc_w, fc_b):
    text = text.astype(jnp.int32).reshape(-1)
    ptab = _tc_project(fc_w, fc_b.reshape(NUM_CLASS, 1), emb_table)
    out = _make_sc_pool()(ptab.reshape(-1), text)
    return out.reshape(NUM_CLASS, BATCH).T
